# baseline (speedup 1.0000x reference)
import functools

import jax
import jax.numpy as jnp
from jax import lax
from jax.experimental import pallas as pl
from jax.experimental.pallas import tpu as pltpu
from jax.experimental.pallas import tpu_sc as plsc

N = 10000
E = 320000
D = 128
NC = 2
NS = 16
C = 128
CPT = ((-(-E // (C * NS)) + 3) // 4) * 4
EPAD = CPT * C * NS
HALF = N // 2
ACC_R = 5120
TRASH = 5100
RPT = ACC_R // NS
WFULL = 320
WLAST = HALF - 15 * RPT

_mesh = plsc.VectorSubcoreMesh(
    core_axis_name="c", subcore_axis_name="s", num_cores=NC, num_subcores=NS)


@functools.partial(
    pl.kernel,
    out_type=jax.ShapeDtypeStruct((N, D), jnp.float32),
    mesh=_mesh,
    scratch_types=[
        pltpu.VMEM((4, C), jnp.int32),
        pltpu.VMEM((4, C), jnp.int32),
        pltpu.VMEM((2, C, D), jnp.float32),
        pltpu.VMEM_SHARED((ACC_R, D), jnp.float32),
        pltpu.SemaphoreType.DMA,
        pltpu.SemaphoreType.DMA,
        pltpu.SemaphoreType.DMA,
        pltpu.SemaphoreType.DMA,
        pltpu.SemaphoreType.DMA,
    ],
)
def _sc_pass(h_hbm, src_hbm, dst2_hbm, out_hbm, sidx, didx, rows, acc, sem,
             sems0, sems1, semi0, semi1):
  cid = lax.axis_index("c")
  sid = lax.axis_index("s")

  def zero(i, _):
    for j in range(D // 16):
      rows[0, i, pl.ds(j * 16, 16)] = jnp.zeros((16,), jnp.float32)
    return 0
  lax.fori_loop(0, C, zero, 0)
  base0 = sid * RPT
  pltpu.sync_copy(rows.at[0], acc.at[pl.ds(base0, C)])
  pltpu.sync_copy(rows.at[0], acc.at[pl.ds(base0 + C, C)])
  pltpu.sync_copy(rows.at[0].at[pl.ds(0, RPT - 2 * C)],
                  acc.at[pl.ds(base0 + 2 * C, RPT - 2 * C)])
  plsc.subcore_barrier()

  ssem = (sems0, sems1)
  isem = (semi0, semi1)

  def load(j, k, p):
    base = (j * NS + sid) * C
    pltpu.async_copy(src_hbm.at[pl.ds(base, C)], sidx.at[k], isem[p])
    pltpu.async_copy(dst2_hbm.at[pl.ds(cid * EPAD + base, C)], didx.at[k],
                     isem[p])

  def wait_i(k, p):
    pltpu.make_async_copy(src_hbm.at[pl.ds(0, C)], sidx.at[k],
                          isem[p]).wait()
    pltpu.make_async_copy(src_hbm.at[pl.ds(0, C)], didx.at[k],
                          isem[p]).wait()

  def fire(k, rb):
    pltpu.async_copy(h_hbm.at[sidx.at[k]], rows.at[rb], sem)

  def wait_g(k, rb):
    pltpu.make_async_copy(h_hbm.at[sidx.at[k]], rows.at[rb], sem).wait()

  def scat(k, rb):
    pltpu.async_copy(rows.at[rb], acc.at[didx.at[k]], ssem[rb], add=True)

  def wait_s(k, rb):
    pltpu.make_async_copy(rows.at[rb], acc.at[didx.at[k]], ssem[rb]).wait()

  load(0, 0, 0)
  load(1, 1, 1)
  wait_i(0, 0); fire(0, 0); load(2, 2, 0)
  wait_g(0, 0); scat(0, 0); wait_i(1, 1); fire(1, 1); load(3, 3, 1)
  wait_g(1, 1); scat(1, 1); wait_i(2, 0); wait_s(0, 0); fire(2, 0)
  load(4, 0, 0)
  wait_g(2, 0); scat(2, 0); wait_i(3, 1); wait_s(1, 1); fire(3, 1)
  load(5, 1, 1)

  def body(t, _):
    for r in range(4):
      j = 4 * t + r
      k, rb, p = r, r % 2, r % 2
      km1, rbm1 = (r - 1) % 4, (r - 1) % 2
      wait_g(km1, rbm1)
      scat(km1, rbm1)
      wait_i(k, p)
      wait_s((r + 2) % 4, rb)
      fire(k, rb)

      @pl.when(j + 2 < CPT)
      def _():
        load(j + 2, (r + 2) % 4, p)
    return 0
  lax.fori_loop(1, CPT // 4, body, 0)

  wait_g(3, 1)
  scat(3, 1)
  wait_s(2, 0)
  wait_s(3, 1)
  plsc.subcore_barrier()

  @pl.when(sid < NS - 1)
  def _():
    pltpu.sync_copy(acc.at[pl.ds(sid * RPT, WFULL)],
                    out_hbm.at[pl.ds(cid * HALF + sid * RPT, WFULL)])

  @pl.when(sid == NS - 1)
  def _():
    pltpu.sync_copy(acc.at[pl.ds((NS - 1) * RPT, WLAST)],
                    out_hbm.at[pl.ds(cid * HALF + (NS - 1) * RPT, WLAST)])


HD = 64


@functools.partial(
    pl.kernel,
    out_type=jax.ShapeDtypeStruct((N, D), jnp.float32),
    mesh=_mesh,
    scratch_types=[
        pltpu.VMEM((C,), jnp.int32),
        pltpu.VMEM((C, D), jnp.float32),
        pltpu.VMEM((C, D), jnp.float32),
        pltpu.VMEM_SHARED((ACC_R, D), jnp.float32),
        pltpu.SemaphoreType.DMA,
    ],
)
def _sc_deg(dst2_hbm, out_hbm, didx, ones_v, zbuf, acc, sem):
  cid = lax.axis_index("c")
  sid = lax.axis_index("s")

  def init(i, _):
    for j in range(D // 16):
      ones_v[i, pl.ds(j * 16, 16)] = jnp.ones((16,), jnp.float32)
      zbuf[i, pl.ds(j * 16, 16)] = jnp.zeros((16,), jnp.float32)
    return 0
  lax.fori_loop(0, C, init, 0)
  base0 = sid * RPT
  pltpu.sync_copy(zbuf, acc.at[pl.ds(base0, C)])
  pltpu.sync_copy(zbuf, acc.at[pl.ds(base0 + C, C)])
  pltpu.sync_copy(zbuf.at[pl.ds(0, RPT - 2 * C)],
                  acc.at[pl.ds(base0 + 2 * C, RPT - 2 * C)])
  plsc.subcore_barrier()

  def body(j, _):
    base = (j * NS + sid) * C
    pltpu.async_copy(dst2_hbm.at[pl.ds(cid * EPAD + base, C)], didx,
                     sem).wait()
    pltpu.sync_copy(ones_v, acc.at[didx], add=True)
    return 0
  lax.fori_loop(0, CPT, body, 0)
  plsc.subcore_barrier()

  @pl.when(sid < NS - 1)
  def _():
    pltpu.sync_copy(acc.at[pl.ds(sid * RPT, WFULL)],
                    out_hbm.at[pl.ds(cid * HALF + sid * RPT, WFULL)])

  @pl.when(sid == NS - 1)
  def _():
    pltpu.sync_copy(acc.at[pl.ds((NS - 1) * RPT, WLAST)],
                    out_hbm.at[pl.ds(cid * HALF + (NS - 1) * RPT, WLAST)])


_BR = 1000


def _row_spec(width):
  return pl.BlockSpec((_BR, width), lambda i: (i, 0))


def _e0_body(deg_ref, feat_ref, h0_ref, s_ref, invd_ref):
  d = jnp.maximum(deg_ref[:, 0], 1.0)[:, None]
  s = lax.rsqrt(d)
  s_ref[...] = s
  invd_ref[...] = 1.0 / d
  h0_ref[...] = feat_ref[...] * s


def _e0(deg, feat):
  return pl.pallas_call(
      _e0_body,
      grid=(N // _BR,),
      in_specs=[_row_spec(D), _row_spec(D)],
      out_specs=[_row_spec(D), _row_spec(1), _row_spec(1)],
      out_shape=[
          jax.ShapeDtypeStruct((N, D), jnp.float32),
          jax.ShapeDtypeStruct((N, 1), jnp.float32),
          jax.ShapeDtypeStruct((N, 1), jnp.float32),
      ],
  )(deg, feat)


def _e1_body(theta_ref, g_ref, feat_ref, s_ref, invd_ref,
             h1_ref, t1_ref, acc_ref):
  g = g_ref[...]
  t1 = -s_ref[...] * g
  t1_ref[...] = t1
  h1_ref[...] = -invd_ref[...] * g
  acc_ref[...] = (theta_ref[0] * feat_ref[...]
                  + theta_ref[1] * (t1 + 1.0) * 0.5)


def _e1(g1, feat, s, invd, theta):
  return pl.pallas_call(
      _e1_body,
      grid=(N // _BR,),
      in_specs=[
          pl.BlockSpec(memory_space=pltpu.SMEM),
          _row_spec(D), _row_spec(D), _row_spec(1), _row_spec(1),
      ],
      out_specs=[_row_spec(D), _row_spec(D), _row_spec(D)],
      out_shape=[
          jax.ShapeDtypeStruct((N, D), jnp.float32),
          jax.ShapeDtypeStruct((N, D), jnp.float32),
          jax.ShapeDtypeStruct((N, D), jnp.float32),
      ],
  )(theta, g1, feat, s, invd)


def _e2_body(theta_ref, g_ref, feat_ref, h0_ref, s_ref, invd_ref, acc_ref,
             h2_ref, acc2_ref):
  g = g_ref[...]
  t2 = -2.0 * s_ref[...] * g - feat_ref[...]
  h2_ref[...] = -2.0 * invd_ref[...] * g - h0_ref[...]
  acc2_ref[...] = acc_ref[...] + theta_ref[2] * (t2 + 1.0) * 0.5


def _e2(g2, feat, h0, s, invd, acc, theta):
  return pl.pallas_call(
      _e2_body,
      grid=(N // _BR,),
      in_specs=[
          pl.BlockSpec(memory_space=pltpu.SMEM),
          _row_spec(D), _row_spec(D), _row_spec(D), _row_spec(1),
          _row_spec(1), _row_spec(D),
      ],
      out_specs=[_row_spec(D), _row_spec(D)],
      out_shape=[
          jax.ShapeDtypeStruct((N, D), jnp.float32),
          jax.ShapeDtypeStruct((N, D), jnp.float32),
      ],
  )(theta, g2, feat, h0, s, invd, acc)


def _e3_body(theta_ref, g_ref, t1_ref, s_ref, acc_ref, w_ref, out_ref):
  g = g_ref[...]
  t3 = -2.0 * s_ref[...] * g - t1_ref[...]
  h = acc_ref[...] + theta_ref[3] * (t3 + 1.0) * 0.5
  y = lax.dot_general(h, w_ref[...], (((1,), (1,)), ((), ())),
                      preferred_element_type=jnp.float32)
  out_ref[...] = jnp.where(y >= 0.0, y, 0.01 * y)


def _e3(g3, t1, s, acc, W, theta):
  return pl.pallas_call(
      _e3_body,
      grid=(N // _BR,),
      in_specs=[
          pl.BlockSpec(memory_space=pltpu.SMEM),
          _row_spec(D), _row_spec(D), _row_spec(1), _row_spec(D),
          pl.BlockSpec((D, D), lambda i: (0, 0)),
      ],
      out_specs=_row_spec(D),
      out_shape=jax.ShapeDtypeStruct((N, D), jnp.float32),
  )(theta, g3, t1, s, acc, W)


def _prep_edges(src, dst):
  pad = EPAD - E
  srcp = jnp.concatenate([src, jnp.zeros((pad,), jnp.int32)])
  padd = jnp.full((pad,), TRASH, jnp.int32)
  d0 = jnp.where(dst < HALF, dst, TRASH)
  d1 = jnp.where(dst >= HALF, dst - HALF, TRASH)
  dst2 = jnp.concatenate([d0, padd, d1, padd])
  return srcp, dst2


def kernel(feat, edge_index, W, theta):
  src = edge_index[0].astype(jnp.int32)
  dst = edge_index[1].astype(jnp.int32)
  theta = theta.astype(jnp.float32)
  srcp, dst2 = _prep_edges(src, dst)

  deg = _sc_deg(dst2)
  h0, s, invd = _e0(deg, feat)
  g1 = _sc_pass(h0, srcp, dst2)
  h1, t1, acc1 = _e1(g1, feat, s, invd, theta)
  g2 = _sc_pass(h1, srcp, dst2)
  h2, acc2 = _e2(g2, feat, h0, s, invd, acc1, theta)
  g3 = _sc_pass(h2, srcp, dst2)
  return _e3(g3, t1, s, acc2, W, theta)


# speedup vs baseline: 1.6165x; 1.6165x over previous
import functools

import jax
import jax.numpy as jnp
from jax import lax
from jax.experimental import pallas as pl
from jax.experimental.pallas import tpu as pltpu
from jax.experimental.pallas import tpu_sc as plsc

N = 10000
E = 320000
D = 128
NC = 2
NS = 16
C = 128
CPT = -(-E // (C * NS))
EPAD = CPT * C * NS
HALF = N // 2
ACC_R = 5120
TRASH = 5100
RPT = ACC_R // NS
WFULL = 320
WLAST = HALF - 15 * RPT

_mesh = plsc.VectorSubcoreMesh(
    core_axis_name="c", subcore_axis_name="s", num_cores=NC, num_subcores=NS)


@functools.partial(
    pl.kernel,
    out_type=jax.ShapeDtypeStruct((N, D), jnp.float32),
    mesh=_mesh,
    scratch_types=[
        pltpu.VMEM((2, C), jnp.int32),
        pltpu.VMEM((2, C), jnp.int32),
        pltpu.VMEM((2, C, D), jnp.float32),
        pltpu.VMEM_SHARED((ACC_R, D), jnp.float32),
        pltpu.SemaphoreType.DMA,
        pltpu.SemaphoreType.DMA,
        pltpu.SemaphoreType.DMA,
    ],
)
def _sc_pass(h_hbm, src_hbm, dst2_hbm, out_hbm, sidx, didx, rows, acc, sem,
             sems0, sems1):
  cid = lax.axis_index("c")
  sid = lax.axis_index("s")

  def zero(i, _):
    for j in range(D // 16):
      rows[0, i, pl.ds(j * 16, 16)] = jnp.zeros((16,), jnp.float32)
    return 0
  lax.fori_loop(0, C, zero, 0)
  base0 = sid * RPT
  pltpu.sync_copy(rows.at[0], acc.at[pl.ds(base0, C)])
  pltpu.sync_copy(rows.at[0], acc.at[pl.ds(base0 + C, C)])
  pltpu.sync_copy(rows.at[0].at[pl.ds(0, RPT - 2 * C)],
                  acc.at[pl.ds(base0 + 2 * C, RPT - 2 * C)])
  plsc.subcore_barrier()

  def start(chunk, b):
    base = (chunk * NS + sid) * C
    pltpu.sync_copy(src_hbm.at[pl.ds(base, C)], sidx.at[b])
    pltpu.sync_copy(dst2_hbm.at[pl.ds(cid * EPAD + base, C)], didx.at[b])
    pltpu.async_copy(h_hbm.at[sidx.at[b]], rows.at[b], sem)

  def wait_g(b):
    pltpu.make_async_copy(h_hbm.at[sidx.at[b]], rows.at[b], sem).wait()

  ssem = (sems0, sems1)

  def scat(b):
    pltpu.async_copy(rows.at[b], acc.at[didx.at[b]], ssem[b], add=True)

  def wait_s(b):
    pltpu.make_async_copy(rows.at[b], acc.at[didx.at[b]], ssem[b]).wait()

  start(0, 0)
  wait_g(0)
  start(1, 1)
  scat(0)

  def body(jj, _):
    wait_g(1)
    wait_s(0)
    start(2 * jj + 2, 0)
    scat(1)
    wait_g(0)
    wait_s(1)

    @pl.when(2 * jj + 3 < CPT)
    def _():
      start(2 * jj + 3, 1)
    scat(0)
    return 0
  lax.fori_loop(0, (CPT - 1) // 2, body, 0)
  wait_s(0)
  plsc.subcore_barrier()

  @pl.when(sid < NS - 1)
  def _():
    pltpu.sync_copy(acc.at[pl.ds(sid * RPT, WFULL)],
                    out_hbm.at[pl.ds(cid * HALF + sid * RPT, WFULL)])

  @pl.when(sid == NS - 1)
  def _():
    pltpu.sync_copy(acc.at[pl.ds((NS - 1) * RPT, WLAST)],
                    out_hbm.at[pl.ds(cid * HALF + (NS - 1) * RPT, WLAST)])


HD = 64


@functools.partial(
    pl.kernel,
    out_type=jax.ShapeDtypeStruct((N, D), jnp.float32),
    mesh=_mesh,
    scratch_types=[
        pltpu.VMEM((C,), jnp.int32),
        pltpu.VMEM((C, D), jnp.float32),
        pltpu.VMEM((C, D), jnp.float32),
        pltpu.VMEM_SHARED((ACC_R, D), jnp.float32),
        pltpu.SemaphoreType.DMA,
    ],
)
def _sc_deg(dst2_hbm, out_hbm, didx, ones_v, zbuf, acc, sem):
  cid = lax.axis_index("c")
  sid = lax.axis_index("s")

  def init(i, _):
    for j in range(D // 16):
      ones_v[i, pl.ds(j * 16, 16)] = jnp.ones((16,), jnp.float32)
      zbuf[i, pl.ds(j * 16, 16)] = jnp.zeros((16,), jnp.float32)
    return 0
  lax.fori_loop(0, C, init, 0)
  base0 = sid * RPT
  pltpu.sync_copy(zbuf, acc.at[pl.ds(base0, C)])
  pltpu.sync_copy(zbuf, acc.at[pl.ds(base0 + C, C)])
  pltpu.sync_copy(zbuf.at[pl.ds(0, RPT - 2 * C)],
                  acc.at[pl.ds(base0 + 2 * C, RPT - 2 * C)])
  plsc.subcore_barrier()

  def body(j, _):
    base = (j * NS + sid) * C
    pltpu.async_copy(dst2_hbm.at[pl.ds(cid * EPAD + base, C)], didx,
                     sem).wait()
    pltpu.sync_copy(ones_v, acc.at[didx], add=True)
    return 0
  lax.fori_loop(0, CPT, body, 0)
  plsc.subcore_barrier()

  @pl.when(sid < NS - 1)
  def _():
    pltpu.sync_copy(acc.at[pl.ds(sid * RPT, WFULL)],
                    out_hbm.at[pl.ds(cid * HALF + sid * RPT, WFULL)])

  @pl.when(sid == NS - 1)
  def _():
    pltpu.sync_copy(acc.at[pl.ds((NS - 1) * RPT, WLAST)],
                    out_hbm.at[pl.ds(cid * HALF + (NS - 1) * RPT, WLAST)])


_BR = 1000


def _row_spec(width):
  return pl.BlockSpec((_BR, width), lambda i: (i, 0))


def _e0_body(deg_ref, feat_ref, h0_ref, s_ref, invd_ref):
  d = jnp.maximum(deg_ref[:, 0], 1.0)[:, None]
  s = lax.rsqrt(d)
  s_ref[...] = s
  invd_ref[...] = 1.0 / d
  h0_ref[...] = feat_ref[...] * s


def _e0(deg, feat):
  return pl.pallas_call(
      _e0_body,
      grid=(N // _BR,),
      in_specs=[_row_spec(D), _row_spec(D)],
      out_specs=[_row_spec(D), _row_spec(1), _row_spec(1)],
      out_shape=[
          jax.ShapeDtypeStruct((N, D), jnp.float32),
          jax.ShapeDtypeStruct((N, 1), jnp.float32),
          jax.ShapeDtypeStruct((N, 1), jnp.float32),
      ],
  )(deg, feat)


def _e1_body(theta_ref, g_ref, feat_ref, s_ref, invd_ref,
             h1_ref, t1_ref, acc_ref):
  g = g_ref[...]
  t1 = -s_ref[...] * g
  t1_ref[...] = t1
  h1_ref[...] = -invd_ref[...] * g
  acc_ref[...] = (theta_ref[0] * feat_ref[...]
                  + theta_ref[1] * (t1 + 1.0) * 0.5)


def _e1(g1, feat, s, invd, theta):
  return pl.pallas_call(
      _e1_body,
      grid=(N // _BR,),
      in_specs=[
          pl.BlockSpec(memory_space=pltpu.SMEM),
          _row_spec(D), _row_spec(D), _row_spec(1), _row_spec(1),
      ],
      out_specs=[_row_spec(D), _row_spec(D), _row_spec(D)],
      out_shape=[
          jax.ShapeDtypeStruct((N, D), jnp.float32),
          jax.ShapeDtypeStruct((N, D), jnp.float32),
          jax.ShapeDtypeStruct((N, D), jnp.float32),
      ],
  )(theta, g1, feat, s, invd)


def _e2_body(theta_ref, g_ref, feat_ref, h0_ref, s_ref, invd_ref, acc_ref,
             h2_ref, acc2_ref):
  g = g_ref[...]
  t2 = -2.0 * s_ref[...] * g - feat_ref[...]
  h2_ref[...] = -2.0 * invd_ref[...] * g - h0_ref[...]
  acc2_ref[...] = acc_ref[...] + theta_ref[2] * (t2 + 1.0) * 0.5


def _e2(g2, feat, h0, s, invd, acc, theta):
  return pl.pallas_call(
      _e2_body,
      grid=(N // _BR,),
      in_specs=[
          pl.BlockSpec(memory_space=pltpu.SMEM),
          _row_spec(D), _row_spec(D), _row_spec(D), _row_spec(1),
          _row_spec(1), _row_spec(D),
      ],
      out_specs=[_row_spec(D), _row_spec(D)],
      out_shape=[
          jax.ShapeDtypeStruct((N, D), jnp.float32),
          jax.ShapeDtypeStruct((N, D), jnp.float32),
      ],
  )(theta, g2, feat, h0, s, invd, acc)


def _e3_body(theta_ref, g_ref, t1_ref, s_ref, acc_ref, w_ref, out_ref):
  g = g_ref[...]
  t3 = -2.0 * s_ref[...] * g - t1_ref[...]
  h = acc_ref[...] + theta_ref[3] * (t3 + 1.0) * 0.5
  y = lax.dot_general(h, w_ref[...], (((1,), (1,)), ((), ())),
                      preferred_element_type=jnp.float32)
  out_ref[...] = jnp.where(y >= 0.0, y, 0.01 * y)


def _e3(g3, t1, s, acc, W, theta):
  return pl.pallas_call(
      _e3_body,
      grid=(N // _BR,),
      in_specs=[
          pl.BlockSpec(memory_space=pltpu.SMEM),
          _row_spec(D), _row_spec(D), _row_spec(1), _row_spec(D),
          pl.BlockSpec((D, D), lambda i: (0, 0)),
      ],
      out_specs=_row_spec(D),
      out_shape=jax.ShapeDtypeStruct((N, D), jnp.float32),
  )(theta, g3, t1, s, acc, W)


def _prep_edges(src, dst):
  pad = EPAD - E
  srcp = jnp.concatenate([src, jnp.zeros((pad,), jnp.int32)])
  padd = jnp.full((pad,), TRASH, jnp.int32)
  d0 = jnp.where(dst < HALF, dst, TRASH)
  d1 = jnp.where(dst >= HALF, dst - HALF, TRASH)
  dst2 = jnp.concatenate([d0, padd, d1, padd])
  return srcp, dst2


def kernel(feat, edge_index, W, theta):
  src = edge_index[0].astype(jnp.int32)
  dst = edge_index[1].astype(jnp.int32)
  theta = theta.astype(jnp.float32)
  srcp, dst2 = _prep_edges(src, dst)

  deg = _sc_deg(dst2)
  h0, s, invd = _e0(deg, feat)
  g1 = _sc_pass(h0, srcp, dst2)
  h1, t1, acc1 = _e1(g1, feat, s, invd, theta)
  g2 = _sc_pass(h1, srcp, dst2)
  h2, acc2 = _e2(g2, feat, h0, s, invd, acc1, theta)
  g3 = _sc_pass(h2, srcp, dst2)
  return _e3(g3, t1, s, acc2, W, theta)


# speedup vs baseline: 1.6170x; 1.0003x over previous
import functools

import jax
import jax.numpy as jnp
from jax import lax
from jax.experimental import pallas as pl
from jax.experimental.pallas import tpu as pltpu
from jax.experimental.pallas import tpu_sc as plsc

N = 10000
E = 320000
D = 128
NC = 2
NS = 16
C = 128
CPT = -(-E // (C * NS))
EPAD = CPT * C * NS
HALF = N // 2
ACC_R = 5120
TRASH = 5100
RPT = ACC_R // NS
WFULL = 320
WLAST = HALF - 15 * RPT

_mesh = plsc.VectorSubcoreMesh(
    core_axis_name="c", subcore_axis_name="s", num_cores=NC, num_subcores=NS)


@functools.partial(
    pl.kernel,
    out_type=jax.ShapeDtypeStruct((N, D), jnp.float32),
    mesh=_mesh,
    scratch_types=[
        pltpu.VMEM((2, C), jnp.int32),
        pltpu.VMEM((2, C), jnp.int32),
        pltpu.VMEM((2, C, D), jnp.float32),
        pltpu.VMEM_SHARED((ACC_R, D), jnp.float32),
        pltpu.SemaphoreType.DMA,
        pltpu.SemaphoreType.DMA,
        pltpu.SemaphoreType.DMA,
    ],
)
def _sc_pass(h_hbm, src_hbm, dst2_hbm, out_hbm, sidx, didx, rows, acc, sem,
             sems0, sems1):
  cid = lax.axis_index("c")
  sid = lax.axis_index("s")

  def zero(i, _):
    for j in range(D // 16):
      rows[0, i, pl.ds(j * 16, 16)] = jnp.zeros((16,), jnp.float32)
    return 0
  lax.fori_loop(0, C, zero, 0)
  base0 = sid * RPT
  pltpu.sync_copy(rows.at[0], acc.at[pl.ds(base0, C)])
  pltpu.sync_copy(rows.at[0], acc.at[pl.ds(base0 + C, C)])
  pltpu.sync_copy(rows.at[0].at[pl.ds(0, RPT - 2 * C)],
                  acc.at[pl.ds(base0 + 2 * C, RPT - 2 * C)])
  plsc.subcore_barrier()

  def start(chunk, b):
    base = (chunk * NS + sid) * C
    pltpu.sync_copy(src_hbm.at[pl.ds(base, C)], sidx.at[b])
    pltpu.sync_copy(dst2_hbm.at[pl.ds(cid * EPAD + base, C)], didx.at[b])
    pltpu.async_copy(h_hbm.at[sidx.at[b]], rows.at[b], sem)

  def wait_g(b):
    pltpu.make_async_copy(h_hbm.at[sidx.at[b]], rows.at[b], sem).wait()

  ssem = (sems0, sems1)

  def scat(b):
    pltpu.async_copy(rows.at[b], acc.at[didx.at[b]], ssem[b], add=True)

  def wait_s(b):
    pltpu.make_async_copy(rows.at[b], acc.at[didx.at[b]], ssem[b]).wait()

  start(0, 0)
  wait_g(0)
  start(1, 1)
  scat(0)

  def body(jj, _):
    wait_g(1)
    wait_s(0)
    start(2 * jj + 2, 0)
    scat(1)
    wait_g(0)
    wait_s(1)

    @pl.when(2 * jj + 3 < CPT)
    def _():
      start(2 * jj + 3, 1)
    scat(0)
    return 0
  lax.fori_loop(0, (CPT - 1) // 2, body, 0)
  wait_s(0)
  plsc.subcore_barrier()

  @pl.when(sid < NS - 1)
  def _():
    pltpu.sync_copy(acc.at[pl.ds(sid * RPT, WFULL)],
                    out_hbm.at[pl.ds(cid * HALF + sid * RPT, WFULL)])

  @pl.when(sid == NS - 1)
  def _():
    pltpu.sync_copy(acc.at[pl.ds((NS - 1) * RPT, WLAST)],
                    out_hbm.at[pl.ds(cid * HALF + (NS - 1) * RPT, WLAST)])


@functools.partial(
    pl.kernel,
    out_type=jax.ShapeDtypeStruct((N, D), jnp.float32),
    mesh=_mesh,
    scratch_types=[
        pltpu.VMEM((C,), jnp.int32),
        pltpu.VMEM((C, D), jnp.float32),
        pltpu.VMEM((C, D), jnp.float32),
        pltpu.VMEM_SHARED((ACC_R, D), jnp.float32),
        pltpu.SemaphoreType.DMA,
    ],
)
def _sc_deg(dst2_hbm, out_hbm, didx, ones_v, zbuf, acc, sem):
  cid = lax.axis_index("c")
  sid = lax.axis_index("s")

  def init(i, _):
    for j in range(D // 16):
      ones_v[i, pl.ds(j * 16, 16)] = jnp.ones((16,), jnp.float32)
      zbuf[i, pl.ds(j * 16, 16)] = jnp.zeros((16,), jnp.float32)
    return 0
  lax.fori_loop(0, C, init, 0)
  base0 = sid * RPT
  pltpu.sync_copy(zbuf, acc.at[pl.ds(base0, C)])
  pltpu.sync_copy(zbuf, acc.at[pl.ds(base0 + C, C)])
  pltpu.sync_copy(zbuf.at[pl.ds(0, RPT - 2 * C)],
                  acc.at[pl.ds(base0 + 2 * C, RPT - 2 * C)])
  plsc.subcore_barrier()

  def body(j, _):
    base = (j * NS + sid) * C
    pltpu.async_copy(dst2_hbm.at[pl.ds(cid * EPAD + base, C)], didx,
                     sem).wait()
    pltpu.sync_copy(ones_v, acc.at[didx], add=True)
    return 0
  lax.fori_loop(0, CPT, body, 0)
  plsc.subcore_barrier()

  @pl.when(sid < NS - 1)
  def _():
    pltpu.sync_copy(acc.at[pl.ds(sid * RPT, WFULL)],
                    out_hbm.at[pl.ds(cid * HALF + sid * RPT, WFULL)])

  @pl.when(sid == NS - 1)
  def _():
    pltpu.sync_copy(acc.at[pl.ds((NS - 1) * RPT, WLAST)],
                    out_hbm.at[pl.ds(cid * HALF + (NS - 1) * RPT, WLAST)])


_BR = 1000


def _row_spec(width):
  return pl.BlockSpec((_BR, width), lambda i: (i, 0))


def _e0_body(deg_ref, feat_ref, h0_ref, s_ref, invd_ref):
  d = jnp.maximum(deg_ref[:, 0], 1.0)[:, None]
  s = lax.rsqrt(d)
  s_ref[...] = s
  invd_ref[...] = 1.0 / d
  h0_ref[...] = feat_ref[...] * s


def _e0(deg, feat):
  return pl.pallas_call(
      _e0_body,
      grid=(N // _BR,),
      in_specs=[_row_spec(D), _row_spec(D)],
      out_specs=[_row_spec(D), _row_spec(1), _row_spec(1)],
      out_shape=[
          jax.ShapeDtypeStruct((N, D), jnp.float32),
          jax.ShapeDtypeStruct((N, 1), jnp.float32),
          jax.ShapeDtypeStruct((N, 1), jnp.float32),
      ],
  )(deg, feat)


def _e1_body(theta_ref, g_ref, feat_ref, s_ref, invd_ref,
             h1_ref, t1_ref, acc_ref):
  g = g_ref[...]
  t1 = -s_ref[...] * g
  t1_ref[...] = t1
  h1_ref[...] = -invd_ref[...] * g
  acc_ref[...] = (theta_ref[0] * feat_ref[...]
                  + theta_ref[1] * (t1 + 1.0) * 0.5)


def _e1(g1, feat, s, invd, theta):
  return pl.pallas_call(
      _e1_body,
      grid=(N // _BR,),
      in_specs=[
          pl.BlockSpec(memory_space=pltpu.SMEM),
          _row_spec(D), _row_spec(D), _row_spec(1), _row_spec(1),
      ],
      out_specs=[_row_spec(D), _row_spec(D), _row_spec(D)],
      out_shape=[
          jax.ShapeDtypeStruct((N, D), jnp.float32),
          jax.ShapeDtypeStruct((N, D), jnp.float32),
          jax.ShapeDtypeStruct((N, D), jnp.float32),
      ],
  )(theta, g1, feat, s, invd)


def _e2_body(theta_ref, g_ref, feat_ref, h0_ref, s_ref, invd_ref, acc_ref,
             h2_ref, acc2_ref):
  g = g_ref[...]
  t2 = -2.0 * s_ref[...] * g - feat_ref[...]
  h2_ref[...] = -2.0 * invd_ref[...] * g - h0_ref[...]
  acc2_ref[...] = acc_ref[...] + theta_ref[2] * (t2 + 1.0) * 0.5


def _e2(g2, feat, h0, s, invd, acc, theta):
  return pl.pallas_call(
      _e2_body,
      grid=(N // _BR,),
      in_specs=[
          pl.BlockSpec(memory_space=pltpu.SMEM),
          _row_spec(D), _row_spec(D), _row_spec(D), _row_spec(1),
          _row_spec(1), _row_spec(D),
      ],
      out_specs=[_row_spec(D), _row_spec(D)],
      out_shape=[
          jax.ShapeDtypeStruct((N, D), jnp.float32),
          jax.ShapeDtypeStruct((N, D), jnp.float32),
      ],
  )(theta, g2, feat, h0, s, invd, acc)


def _e3_body(theta_ref, g_ref, t1_ref, s_ref, acc_ref, w_ref, out_ref):
  g = g_ref[...]
  t3 = -2.0 * s_ref[...] * g - t1_ref[...]
  h = acc_ref[...] + theta_ref[3] * (t3 + 1.0) * 0.5
  y = lax.dot_general(h, w_ref[...], (((1,), (1,)), ((), ())),
                      preferred_element_type=jnp.float32)
  out_ref[...] = jnp.where(y >= 0.0, y, 0.01 * y)


def _e3(g3, t1, s, acc, W, theta):
  return pl.pallas_call(
      _e3_body,
      grid=(N // _BR,),
      in_specs=[
          pl.BlockSpec(memory_space=pltpu.SMEM),
          _row_spec(D), _row_spec(D), _row_spec(1), _row_spec(D),
          pl.BlockSpec((D, D), lambda i: (0, 0)),
      ],
      out_specs=_row_spec(D),
      out_shape=jax.ShapeDtypeStruct((N, D), jnp.float32),
  )(theta, g3, t1, s, acc, W)


def _prep_edges(src, dst):
  pad = EPAD - E
  srcp = jnp.concatenate([src, jnp.zeros((pad,), jnp.int32)])
  padd = jnp.full((pad,), TRASH, jnp.int32)
  d0 = jnp.where(dst < HALF, dst, TRASH)
  d1 = jnp.where(dst >= HALF, dst - HALF, TRASH)
  dst2 = jnp.concatenate([d0, padd, d1, padd])
  return srcp, dst2


def kernel(feat, edge_index, W, theta):
  src = edge_index[0].astype(jnp.int32)
  dst = edge_index[1].astype(jnp.int32)
  theta = theta.astype(jnp.float32)
  srcp, dst2 = _prep_edges(src, dst)

  deg = _sc_deg(dst2)
  h0, s, invd = _e0(deg, feat)
  g1 = _sc_pass(h0, srcp, dst2)
  h1, t1, acc1 = _e1(g1, feat, s, invd, theta)
  g2 = _sc_pass(h1, srcp, dst2)
  h2, acc2 = _e2(g2, feat, h0, s, invd, acc1, theta)
  g3 = _sc_pass(h2, srcp, dst2)
  return _e3(g3, t1, s, acc2, W, theta)
